# Initial kernel scaffold; baseline (speedup 1.0000x reference)
import jax
import jax.numpy as jnp
from jax.experimental import pallas as pl


def kernel(feat_idx, W):
    raise NotImplementedError("write your pallas kernel here")


# speedup vs baseline: 121.5694x; 121.5694x over previous
import functools

import jax
import jax.numpy as jnp
from jax import lax
from jax.experimental import pallas as pl
from jax.experimental.pallas import tpu as pltpu
from jax.experimental.pallas import tpu_sc as plsc

_NF = 1_000_000
_B = 16384
_J = 200
_NC = 2
_NS = 16
_NW = _NC * _NS
_L = 16
_NG = _B // _L
_GPW = _NG // _NW
_IPG = _J * _L
_DW = 128
_ND = _IPG // _DW


def _sc_body(ft_hbm, w_hbm, out_hbm, ibuf, vbuf, obuf, sem):
    wid = lax.axis_index("s") * _NC + lax.axis_index("c")

    def group(gl, carry):
        g = wid * _GPW + gl
        pltpu.sync_copy(ft_hbm.at[g], ibuf)
        copies = [
            pltpu.async_copy(w_hbm.at[ibuf.at[d]], vbuf.at[d], sem)
            for d in range(_ND)
        ]
        for c in copies:
            c.wait()
        acc = jnp.zeros((_L,), jnp.float32)
        for j in range(_J):
            flat = j * _L
            acc = acc + vbuf[flat // _DW, pl.ds(flat % _DW, _L)]
        obuf[pl.ds(gl * _L, _L)] = acc
        return carry

    lax.fori_loop(0, _GPW, group, 0)
    pltpu.sync_copy(obuf, out_hbm.at[pl.ds(wid * _GPW * _L, _GPW * _L)])


@functools.partial(
    pl.kernel,
    out_type=jax.ShapeDtypeStruct((_B,), jnp.float32),
    mesh=plsc.VectorSubcoreMesh(core_axis_name="c", subcore_axis_name="s"),
    scratch_types=[
        pltpu.VMEM((_ND, _DW), jnp.int32),
        pltpu.VMEM((_ND, _DW), jnp.float32),
        pltpu.VMEM((_GPW * _L,), jnp.float32),
        pltpu.SemaphoreType.DMA,
    ],
)
def _sc_call(ft_hbm, w_hbm, out_hbm, ibuf, vbuf, obuf, sem):
    _sc_body(ft_hbm, w_hbm, out_hbm, ibuf, vbuf, obuf, sem)


@jax.jit
def kernel(feat_idx, W):
    ft4 = (
        feat_idx.reshape(_NG, _L, _J)
        .transpose(0, 2, 1)
        .reshape(_NG, _ND, _DW)
    )
    w_ext = jnp.concatenate([jnp.zeros((1,), W.dtype), W.reshape(-1)])
    out = _sc_call(ft4, w_ext)
    return out.reshape(_B, 1)


# speedup vs baseline: 141.0016x; 1.1598x over previous
import functools

import jax
import jax.numpy as jnp
from jax import lax
from jax.experimental import pallas as pl
from jax.experimental.pallas import tpu as pltpu
from jax.experimental.pallas import tpu_sc as plsc

_NF = 1_000_000
_B = 16384
_J = 200
_NC = 2
_NS = 16
_NW = _NC * _NS
_L = 16
_NG = _B // _L
_GPW = _NG // _NW
_IPG = _J * _L
_DW = 128
_ND = _IPG // _DW


def _sc_body(ft_hbm, w_hbm, out_hbm, i_a, v_a, i_b, v_b, obuf, sem_a, sem_b):
    wid = lax.axis_index("s") * _NC + lax.axis_index("c")
    base = wid * _GPW

    def fire(g, ibuf, vbuf, sem):
        pltpu.sync_copy(ft_hbm.at[g], ibuf)
        for d in range(_ND):
            pltpu.async_copy(w_hbm.at[ibuf.at[d]], vbuf.at[d], sem)

    def drain(ibuf, vbuf, sem):
        for d in range(_ND):
            pltpu.make_async_copy(w_hbm.at[ibuf.at[d]], vbuf.at[d], sem).wait()

    def accum(vbuf, gl):
        acc = jnp.zeros((_L,), jnp.float32)
        for j in range(_J):
            flat = j * _L
            acc = acc + vbuf[flat // _DW, pl.ds(flat % _DW, _L)]
        obuf[pl.ds(gl * _L, _L)] = acc

    fire(base, i_a, v_a, sem_a)

    def body(k, carry):
        fire(base + 2 * k + 1, i_b, v_b, sem_b)
        drain(i_a, v_a, sem_a)
        accum(v_a, 2 * k)
        fire(base + jnp.minimum(2 * k + 2, _GPW - 1), i_a, v_a, sem_a)
        drain(i_b, v_b, sem_b)
        accum(v_b, 2 * k + 1)
        return carry

    lax.fori_loop(0, _GPW // 2, body, 0)
    drain(i_a, v_a, sem_a)
    pltpu.sync_copy(obuf, out_hbm.at[pl.ds(base * _L, _GPW * _L)])


@functools.partial(
    pl.kernel,
    out_type=jax.ShapeDtypeStruct((_B,), jnp.float32),
    mesh=plsc.VectorSubcoreMesh(core_axis_name="c", subcore_axis_name="s"),
    scratch_types=[
        pltpu.VMEM((_ND, _DW), jnp.int32),
        pltpu.VMEM((_ND, _DW), jnp.float32),
        pltpu.VMEM((_ND, _DW), jnp.int32),
        pltpu.VMEM((_ND, _DW), jnp.float32),
        pltpu.VMEM((_GPW * _L,), jnp.float32),
        pltpu.SemaphoreType.DMA,
        pltpu.SemaphoreType.DMA,
    ],
)
def _sc_call(ft_hbm, w_hbm, out_hbm, i_a, v_a, i_b, v_b, obuf, sem_a, sem_b):
    _sc_body(ft_hbm, w_hbm, out_hbm, i_a, v_a, i_b, v_b, obuf, sem_a, sem_b)


@jax.jit
def kernel(feat_idx, W):
    ft4 = (
        feat_idx.reshape(_NG, _L, _J)
        .transpose(0, 2, 1)
        .reshape(_NG, _ND, _DW)
    )
    w_ext = jnp.concatenate([jnp.zeros((1,), W.dtype), W.reshape(-1)])
    out = _sc_call(ft4, w_ext)
    return out.reshape(_B, 1)


# speedup vs baseline: 144.0527x; 1.0216x over previous
import functools

import jax
import jax.numpy as jnp
from jax import lax
from jax.experimental import pallas as pl
from jax.experimental.pallas import tpu as pltpu
from jax.experimental.pallas import tpu_sc as plsc

_NF = 1_000_000
_B = 16384
_J = 200
_NC = 2
_NS = 16
_NW = _NC * _NS
_L = 16
_NG = _B // _L
_GPW = _NG // _NW
_IPG = _J * _L
_DW = 128
_ND = _IPG // _DW


def _sc_body(ft_hbm, w_hbm, out_hbm, i_a, v_a, i_b, v_b, obuf, sem_a, sem_b):
    wid = lax.axis_index("s") * _NC + lax.axis_index("c")
    base = wid * _GPW

    def fire(g, ibuf, vbuf, sem):
        pltpu.sync_copy(ft_hbm.at[g], ibuf)
        pltpu.async_copy(w_hbm.at[ibuf], vbuf, sem)

    def drain(ibuf, vbuf, sem):
        pltpu.make_async_copy(w_hbm.at[ibuf], vbuf, sem).wait()

    def accum(vbuf, gl):
        acc = jnp.zeros((_L,), jnp.float32)
        for j in range(_J):
            acc = acc + vbuf[pl.ds(j * _L, _L)]
        obuf[pl.ds(gl * _L, _L)] = acc

    fire(base, i_a, v_a, sem_a)

    def body(k, carry):
        fire(base + 2 * k + 1, i_b, v_b, sem_b)
        drain(i_a, v_a, sem_a)
        accum(v_a, 2 * k)
        fire(base + jnp.minimum(2 * k + 2, _GPW - 1), i_a, v_a, sem_a)
        drain(i_b, v_b, sem_b)
        accum(v_b, 2 * k + 1)
        return carry

    lax.fori_loop(0, _GPW // 2, body, 0)
    drain(i_a, v_a, sem_a)
    pltpu.sync_copy(obuf, out_hbm.at[pl.ds(base * _L, _GPW * _L)])


@functools.partial(
    pl.kernel,
    out_type=jax.ShapeDtypeStruct((_B,), jnp.float32),
    mesh=plsc.VectorSubcoreMesh(core_axis_name="c", subcore_axis_name="s"),
    scratch_types=[
        pltpu.VMEM((_IPG,), jnp.int32),
        pltpu.VMEM((_IPG,), jnp.float32),
        pltpu.VMEM((_IPG,), jnp.int32),
        pltpu.VMEM((_IPG,), jnp.float32),
        pltpu.VMEM((_GPW * _L,), jnp.float32),
        pltpu.SemaphoreType.DMA,
        pltpu.SemaphoreType.DMA,
    ],
)
def _sc_call(ft_hbm, w_hbm, out_hbm, i_a, v_a, i_b, v_b, obuf, sem_a, sem_b):
    _sc_body(ft_hbm, w_hbm, out_hbm, i_a, v_a, i_b, v_b, obuf, sem_a, sem_b)


@jax.jit
def kernel(feat_idx, W):
    ft4 = (
        feat_idx.reshape(_NG, _L, _J)
        .transpose(0, 2, 1)
        .reshape(_NG, _IPG)
    )
    w_ext = jnp.concatenate([jnp.zeros((1,), W.dtype), W.reshape(-1)])
    out = _sc_call(ft4, w_ext)
    return out.reshape(_B, 1)
